# baseline (speedup 1.0000x reference)
import functools

import jax
import jax.numpy as jnp
from jax import lax
from jax.experimental import pallas as pl
from jax.experimental.pallas import tpu as pltpu
from jax.experimental.pallas import tpu_sc as plsc

_N_FIELDS = 26
_VOCAB = 100000
_DIM = 64
_BATCH = 16384
_PAIRS = _N_FIELDS * _VOCAB // 2
_NC = 2
_NS = 16
_NW = _NC * _NS
_BPW = _BATCH // _NW
_SLAB = 128
_NPH = _BPW // _SLAB
_LANES = 16
_ACCR_OFF = _NS * _SLAB


def _sc_body(x_hbm, tab_hbm, out_hbm, xbuf, pair0, pair1, ip0, ip1,
             bd0, bd1, zbuf, mstage, shacc,
             gs0, gs1, ss0, ss1):
    c = lax.axis_index("c")
    s = lax.axis_index("s")
    wid = s * _NC + c
    base = wid * _BPW

    ip = (ip0, ip1)
    bd = (bd0, bd1)
    pair = (pair0, pair1)
    gs = (gs0, gs1)
    ss = (ss0, ss1)

    for f in range(_N_FIELDS):
        pltpu.sync_copy(x_hbm.at[pl.ds(f * _BATCH + base, _BPW)],
                        xbuf.at[pl.ds(f * _BPW, _BPW)])

    zeros = jnp.zeros((_LANES,), jnp.float32)

    def zero_body(i, _):
        for j in range(2 * _DIM // _LANES):
            zbuf[i, pl.ds(j * _LANES, _LANES)] = zeros
        return 0
    lax.fori_loop(0, _SLAB, zero_body, 0)

    lane = lax.iota(jnp.int32, _LANES)
    arow = s * _SLAB

    for p in range(_NPH):
        pltpu.sync_copy(zbuf, shacc.at[pl.ds(arow, _SLAB)])
        pltpu.sync_copy(zbuf, shacc.at[pl.ds(_ACCR_OFF + arow, _SLAB)])

        def fill(f, h, p=p):
            def body(i, _):
                sl = pl.ds(i * _LANES, _LANES)
                off = f * _BPW + p * _SLAB + i * _LANES
                v = xbuf[pl.ds(off, _LANES)]
                w = v + f * _VOCAB
                ip[h][sl] = lax.shift_right_logical(w, 1)
                par = lax.rem(w, 2)
                lrow = lane + (arow + i * _LANES)
                bd[h][sl] = lrow + par * _ACCR_OFF
                return 0
            lax.fori_loop(0, _SLAB // _LANES, body, 0)

        def loop_k(k, _):
            for h in range(2):
                f = 2 * k + h
                @pl.when(k > 0)
                def _():
                    pltpu.make_async_copy(
                        pair[h], shacc.at[bd[h]], ss[h]).wait()
                fill(f, h)
                pltpu.async_copy(tab_hbm.at[ip[h]], pair[h], gs[h])
            for h in range(2):
                pltpu.make_async_copy(tab_hbm.at[ip[h]], pair[h], gs[h]).wait()
                pltpu.async_copy(pair[h], shacc.at[bd[h]], ss[h], add=True)
            return 0

        lax.fori_loop(0, _N_FIELDS // 2, loop_k, 0)

        for h in range(2):
            pltpu.make_async_copy(pair[h], shacc.at[bd[h]], ss[h]).wait()

        pltpu.sync_copy(shacc.at[pl.ds(arow, _SLAB)], pair0)
        pltpu.sync_copy(shacc.at[pl.ds(_ACCR_OFF + arow, _SLAB)], pair1)

        def merge_body(i, _):
            for j in range(_DIM // _LANES):
                sl = pl.ds(j * _LANES, _LANES)
                mstage[pl.ds(i * _DIM + j * _LANES, _LANES)] = (
                    pair0[i, sl] + pair1[i, pl.ds(_DIM + j * _LANES, _LANES)])
            return 0
        lax.fori_loop(0, _SLAB, merge_body, 0)
        out_off = (base + p * _SLAB) * _DIM
        pltpu.sync_copy(mstage, out_hbm.at[pl.ds(out_off, _SLAB * _DIM)])


@jax.jit
def kernel(x, tables):
    x1 = x.T.reshape(_N_FIELDS * _BATCH)
    tab = tables.reshape(_PAIRS, 2 * _DIM)
    scratch = [
        pltpu.VMEM((_N_FIELDS * _BPW,), jnp.int32),
        pltpu.VMEM((_SLAB, 2 * _DIM), jnp.float32),
        pltpu.VMEM((_SLAB, 2 * _DIM), jnp.float32),
        pltpu.VMEM((_SLAB,), jnp.int32),
        pltpu.VMEM((_SLAB,), jnp.int32),
        pltpu.VMEM((_SLAB,), jnp.int32),
        pltpu.VMEM((_SLAB,), jnp.int32),
        pltpu.VMEM((_SLAB, 2 * _DIM), jnp.float32),
        pltpu.VMEM((_SLAB * _DIM,), jnp.float32),
        pltpu.VMEM_SHARED((2 * _NS * _SLAB, 2 * _DIM),
                          jnp.float32),
        pltpu.SemaphoreType.DMA,
        pltpu.SemaphoreType.DMA,
        pltpu.SemaphoreType.DMA,
        pltpu.SemaphoreType.DMA,
    ]
    run = functools.partial(
        pl.kernel,
        out_type=jax.ShapeDtypeStruct((_BATCH * _DIM,), jnp.float32),
        mesh=plsc.VectorSubcoreMesh(core_axis_name="c", subcore_axis_name="s"),
        scratch_types=scratch,
    )(_sc_body)
    return run(x1, tab).reshape(_BATCH, _DIM)


# speedup vs baseline: 1.0027x; 1.0027x over previous
import functools

import jax
import jax.numpy as jnp
from jax import lax
from jax.experimental import pallas as pl
from jax.experimental.pallas import tpu as pltpu
from jax.experimental.pallas import tpu_sc as plsc

_N_FIELDS = 26
_VOCAB = 100000
_DIM = 64
_BATCH = 16384
_PAIRS = _N_FIELDS * _VOCAB // 2
_NC = 2
_NS = 16
_NW = _NC * _NS
_BPW = _BATCH // _NW
_SLAB = 128
_NPH = _BPW // _SLAB
_LANES = 16
_ACCR_OFF = _NS * _SLAB


def _sc_body(x_hbm, tab_hbm, out_hbm, xbuf, pair0, pair1, ip0, ip1,
             bd0, bd1, zbuf, mstage, shacc,
             gs0, gs1, ss0, ss1):
    c = lax.axis_index("c")
    s = lax.axis_index("s")
    wid = s * _NC + c
    base = wid * _BPW

    tabp_hbm = tab_hbm.reshape(_PAIRS, 2 * _DIM)

    ip = (ip0, ip1)
    bd = (bd0, bd1)
    pair = (pair0, pair1)
    gs = (gs0, gs1)
    ss = (ss0, ss1)

    for f in range(_N_FIELDS):
        pltpu.sync_copy(x_hbm.at[pl.ds(f * _BATCH + base, _BPW)],
                        xbuf.at[pl.ds(f * _BPW, _BPW)])

    zeros = jnp.zeros((_LANES,), jnp.float32)

    def zero_body(i, _):
        for j in range(2 * _DIM // _LANES):
            zbuf[i, pl.ds(j * _LANES, _LANES)] = zeros
        return 0
    lax.fori_loop(0, _SLAB, zero_body, 0)

    lane = lax.iota(jnp.int32, _LANES)
    arow = s * _SLAB

    for p in range(_NPH):
        pltpu.sync_copy(zbuf, shacc.at[pl.ds(arow, _SLAB)])
        pltpu.sync_copy(zbuf, shacc.at[pl.ds(_ACCR_OFF + arow, _SLAB)])

        def fill(f, h, p=p):
            def body(i, _):
                sl = pl.ds(i * _LANES, _LANES)
                off = f * _BPW + p * _SLAB + i * _LANES
                v = xbuf[pl.ds(off, _LANES)]
                w = v + f * _VOCAB
                ip[h][sl] = lax.shift_right_logical(w, 1)
                par = lax.rem(w, 2)
                lrow = lane + (arow + i * _LANES)
                bd[h][sl] = lrow + par * _ACCR_OFF
                return 0
            lax.fori_loop(0, _SLAB // _LANES, body, 0)

        def loop_k(k, _):
            for h in range(2):
                f = 2 * k + h
                @pl.when(k > 0)
                def _():
                    pltpu.make_async_copy(
                        pair[h], shacc.at[bd[h]], ss[h]).wait()
                fill(f, h)
                pltpu.async_copy(tabp_hbm.at[ip[h]], pair[h], gs[h])
            for h in range(2):
                pltpu.make_async_copy(tabp_hbm.at[ip[h]], pair[h],
                                      gs[h]).wait()
                pltpu.async_copy(pair[h], shacc.at[bd[h]], ss[h], add=True)
            return 0

        lax.fori_loop(0, _N_FIELDS // 2, loop_k, 0)

        for h in range(2):
            pltpu.make_async_copy(pair[h], shacc.at[bd[h]], ss[h]).wait()

        pltpu.sync_copy(shacc.at[pl.ds(arow, _SLAB)], pair0)
        pltpu.sync_copy(shacc.at[pl.ds(_ACCR_OFF + arow, _SLAB)], pair1)

        def merge_body(i, _):
            for j in range(_DIM // _LANES):
                sl = pl.ds(j * _LANES, _LANES)
                mstage[pl.ds(i * _DIM + j * _LANES, _LANES)] = (
                    pair0[i, sl] + pair1[i, pl.ds(_DIM + j * _LANES, _LANES)])
            return 0
        lax.fori_loop(0, _SLAB, merge_body, 0)
        out_off = (base + p * _SLAB) * _DIM
        pltpu.sync_copy(mstage, out_hbm.at[pl.ds(out_off, _SLAB * _DIM)])


@jax.jit
def kernel(x, tables):
    x1 = x.T.reshape(_N_FIELDS * _BATCH)
    tab = tables.reshape(_N_FIELDS, _VOCAB // 2, 2 * _DIM)
    scratch = [
        pltpu.VMEM((_N_FIELDS * _BPW,), jnp.int32),
        pltpu.VMEM((_SLAB, 2 * _DIM), jnp.float32),
        pltpu.VMEM((_SLAB, 2 * _DIM), jnp.float32),
        pltpu.VMEM((_SLAB,), jnp.int32),
        pltpu.VMEM((_SLAB,), jnp.int32),
        pltpu.VMEM((_SLAB,), jnp.int32),
        pltpu.VMEM((_SLAB,), jnp.int32),
        pltpu.VMEM((_SLAB, 2 * _DIM), jnp.float32),
        pltpu.VMEM((_SLAB * _DIM,), jnp.float32),
        pltpu.VMEM_SHARED((2 * _NS * _SLAB, 2 * _DIM),
                          jnp.float32),
        pltpu.SemaphoreType.DMA,
        pltpu.SemaphoreType.DMA,
        pltpu.SemaphoreType.DMA,
        pltpu.SemaphoreType.DMA,
    ]
    run = functools.partial(
        pl.kernel,
        out_type=jax.ShapeDtypeStruct((_BATCH * _DIM,), jnp.float32),
        mesh=plsc.VectorSubcoreMesh(core_axis_name="c", subcore_axis_name="s"),
        scratch_types=scratch,
    )(_sc_body)
    return run(x1, tab).reshape(_BATCH, _DIM)
